# baseline (speedup 1.0000x reference)
import functools

import jax
import jax.numpy as jnp
from jax import lax
from jax.experimental import pallas as pl
from jax.experimental.pallas import tpu as pltpu
from jax.experimental.pallas import tpu_sc as plsc

_SCALE = 8.0


@functools.lru_cache(maxsize=None)
def _make_kernel(b_flat, vocab, d):
    info = plsc.get_sparse_core_info()
    nw = info.num_cores * info.num_subcores
    lanes = info.num_lanes
    assert b_flat % nw == 0
    b_per_w = b_flat // nw
    chunk = 800
    assert b_per_w % chunk == 0
    n_chunks = b_per_w // chunk
    assert d % lanes == 0
    d_vecs = d // lanes

    mesh = plsc.VectorSubcoreMesh(core_axis_name="c", subcore_axis_name="s")

    @functools.partial(
        pl.kernel,
        mesh=mesh,
        out_type=jax.ShapeDtypeStruct((b_flat, d), jnp.float32),
        scratch_types=[
            pltpu.VMEM((b_per_w,), jnp.int32),
            pltpu.VMEM((chunk, d), jnp.float32),
            pltpu.SemaphoreType.DMA,
        ],
        compiler_params=pltpu.CompilerParams(use_tc_tiling_on_sc=False),
    )
    def k(table_hbm, idx_hbm, out_hbm, idx_v, rows_v, sem):
        wid = lax.axis_index("s") * info.num_cores + lax.axis_index("c")
        base = wid * b_per_w
        pltpu.sync_copy(idx_hbm.at[pl.ds(base, b_per_w)], idx_v)

        def chunk_body(c, carry):
            pltpu.async_copy(
                table_hbm.at[idx_v.at[pl.ds(c * chunk, chunk)]], rows_v, sem
            ).wait()

            def row_body(r, rcarry):
                for dv in range(d_vecs):
                    sl = pl.ds(dv * lanes, lanes)
                    rows_v[r, sl] = rows_v[r, sl] * _SCALE
                return rcarry

            lax.fori_loop(0, chunk, row_body, 0)
            pltpu.sync_copy(rows_v, out_hbm.at[pl.ds(base + c * chunk, chunk)])
            return carry

        lax.fori_loop(0, n_chunks, chunk_body, 0)

    return k


def kernel(x, embedding_weight):
    b, s = x.shape
    vocab, d = embedding_weight.shape
    idx = x.reshape(b * s)
    k = _make_kernel(b * s, vocab, d)
    out = k(embedding_weight, idx)
    return out.reshape(b, s, d)


# speedup vs baseline: 1.0398x; 1.0398x over previous
import functools

import jax
import jax.numpy as jnp
from jax import lax
from jax.experimental import pallas as pl
from jax.experimental.pallas import tpu as pltpu
from jax.experimental.pallas import tpu_sc as plsc

_SCALE = 8.0
_NBUF = 4


@functools.lru_cache(maxsize=None)
def _make_kernel(b_flat, vocab, d):
    info = plsc.get_sparse_core_info()
    nw = info.num_cores * info.num_subcores
    lanes = info.num_lanes
    assert b_flat % nw == 0
    b_per_w = b_flat // nw
    chunk = 400
    assert b_per_w % (chunk * _NBUF) == 0
    n_chunks = b_per_w // chunk
    n_outer = n_chunks // _NBUF
    assert d % lanes == 0
    d_vecs = d // lanes
    rows_unroll = 4
    assert chunk % rows_unroll == 0

    mesh = plsc.VectorSubcoreMesh(core_axis_name="c", subcore_axis_name="s")

    @functools.partial(
        pl.kernel,
        mesh=mesh,
        out_type=jax.ShapeDtypeStruct((b_flat, d), jnp.float32),
        scratch_types=[
            pltpu.VMEM((b_per_w,), jnp.int32),
            [pltpu.VMEM((chunk, d), jnp.float32) for _ in range(_NBUF)],
            [pltpu.SemaphoreType.DMA for _ in range(_NBUF)],
            [pltpu.SemaphoreType.DMA for _ in range(_NBUF)],
        ],
        compiler_params=pltpu.CompilerParams(use_tc_tiling_on_sc=False),
    )
    def k(table_hbm, idx_hbm, out_hbm, idx_v, bufs, gsems, ssems):
        wid = lax.axis_index("s") * info.num_cores + lax.axis_index("c")
        base = wid * b_per_w
        pltpu.sync_copy(idx_hbm.at[pl.ds(base, b_per_w)], idx_v)

        def gather_start(c, buf, sem):
            pltpu.async_copy(
                table_hbm.at[idx_v.at[pl.ds(c * chunk, chunk)]], buf, sem
            )

        def gather_wait(buf, sem):
            pltpu.make_async_copy(
                table_hbm.at[pl.ds(0, chunk)], buf, sem
            ).wait()

        def store_start(c, buf, sem):
            pltpu.async_copy(
                buf, out_hbm.at[pl.ds(base + c * chunk, chunk)], sem
            )

        def store_wait(buf, sem):
            pltpu.make_async_copy(
                buf, out_hbm.at[pl.ds(base, chunk)], sem
            ).wait()

        def scale(buf):
            def strided_body(i, carry):
                r = i * rows_unroll
                for ru in range(rows_unroll):
                    for dv in range(d_vecs):
                        sl = pl.ds(dv * lanes, lanes)
                        buf[r + ru, sl] = buf[r + ru, sl] * _SCALE
                return carry

            lax.fori_loop(0, chunk // rows_unroll, strided_body, 0)

        for j in range(_NBUF - 1):
            gather_start(j, bufs[j], gsems[j])

        def outer_body(p, carry):
            for j in range(_NBUF):
                c = p * _NBUF + j
                pc = c + _NBUF - 1
                pj = (j + _NBUF - 1) % _NBUF

                @pl.when(jnp.logical_and(pc >= _NBUF, pc < n_chunks))
                def _():
                    store_wait(bufs[pj], ssems[pj])

                @pl.when(pc < n_chunks)
                def _():
                    gather_start(pc, bufs[pj], gsems[pj])

                gather_wait(bufs[j], gsems[j])
                scale(bufs[j])
                store_start(c, bufs[j], ssems[j])
            return carry

        lax.fori_loop(0, n_outer, outer_body, 0)

        for j in range(_NBUF):
            store_wait(bufs[j], ssems[j])

    return k


def kernel(x, embedding_weight):
    b, s = x.shape
    vocab, d = embedding_weight.shape
    idx = x.reshape(b * s)
    k = _make_kernel(b * s, vocab, d)
    out = k(embedding_weight, idx)
    return out.reshape(b, s, d)
